# baseline (speedup 1.0000x reference)
import functools

import jax
import jax.numpy as jnp
from jax import lax
from jax.experimental import pallas as pl
from jax.experimental.pallas import tpu as pltpu

_LO = 256
_E_TILE = 4096
_N_PAR = 2


def _round_up(x, m):
    return ((x + m - 1) // m) * m


def _scatter_kernel(idx_ref, ev_ref, out_ref, *, hi_radix):
    ei = pl.program_id(1)

    @pl.when(ei == 0)
    def _init():
        out_ref[...] = jnp.zeros_like(out_ref)

    e_tile = ev_ref.shape[1]

    ev = ev_ref[...]
    d = ev + jnp.cos(ev)
    d16 = d.astype(jnp.bfloat16)

    hi_iota = lax.broadcasted_iota(jnp.int32, (hi_radix, e_tile), 0)
    lo_iota = lax.broadcasted_iota(jnp.int32, (_LO, e_tile), 0)

    accs = []
    for s in (0, 1):
        nid = idx_ref[s:s + 1, :]
        h = nid >> 8
        l = nid & (_LO - 1)
        hmask = (hi_iota == h).astype(jnp.bfloat16)
        lmask = (lo_iota == l).astype(jnp.bfloat16)
        dd = d16 if s == 0 else -d16
        for r in range(3):
            mm = lax.dot_general(
                hmask * dd[r:r + 1, :], lmask, (((1,), (1,)), ((), ())),
                preferred_element_type=jnp.float32)
            accs.append(mm)
    out_ref[0 * hi_radix:1 * hi_radix, :] += accs[0] + accs[3]
    out_ref[1 * hi_radix:2 * hi_radix, :] += accs[1] + accs[4]
    out_ref[2 * hi_radix:3 * hi_radix, :] += accs[2] + accs[5]


def kernel(node_feature, node_feature_ghost, edge_vec, edge_idx):
    tot_num = node_feature.shape[0] + node_feature_ghost.shape[0]
    num_edges = edge_vec.shape[0]

    n_pad = _round_up(max(tot_num, 1), _LO)
    hi_radix = n_pad // _LO

    e_tile = min(_E_TILE, _round_up(max(num_edges, 1), 256))
    e_pad = _round_up(max(num_edges, 1), _N_PAR * e_tile)
    e_blocks = e_pad // e_tile
    epc = e_blocks // _N_PAR

    if e_pad == num_edges:
        idx = edge_idx.astype(jnp.int32)
    else:
        idx = jnp.full((2, e_pad), -1, jnp.int32)
        idx = idx.at[:, :num_edges].set(edge_idx.astype(jnp.int32))

    if e_pad == num_edges:
        ev_t = edge_vec.T.astype(jnp.float32)
    else:
        ev_t = jnp.zeros((3, e_pad), jnp.float32)
        ev_t = ev_t.at[:, :num_edges].set(edge_vec.T.astype(jnp.float32))

    out = pl.pallas_call(
        functools.partial(_scatter_kernel, hi_radix=hi_radix),
        out_shape=jax.ShapeDtypeStruct((_N_PAR * 3 * hi_radix, _LO), jnp.float32),
        grid=(_N_PAR, epc),
        in_specs=[
            pl.BlockSpec((2, e_tile), lambda p, ei: (0, p * epc + ei)),
            pl.BlockSpec((3, e_tile), lambda p, ei: (0, p * epc + ei)),
        ],
        out_specs=pl.BlockSpec((3 * hi_radix, _LO), lambda p, ei: (p, 0)),
        compiler_params=pltpu.CompilerParams(
            dimension_semantics=("parallel", "arbitrary"),
            vmem_limit_bytes=64 * 1024 * 1024,
        ),
        cost_estimate=pl.CostEstimate(
            flops=2 * 2 * 2 * 3 * hi_radix * _LO * e_pad,
            transcendentals=3 * e_pad,
            bytes_accessed=(idx.size + ev_t.size) * 4 + 2 * 3 * n_pad * 4,
        ),
    )(idx, ev_t)

    force_t = out.reshape(_N_PAR, 3, hi_radix, _LO).sum(axis=0)
    force = force_t.transpose(1, 2, 0).reshape(n_pad, 3)[:tot_num]

    return {
        "node_feature": node_feature,
        "node_feature_ghost": node_feature_ghost,
        "edge_vec": edge_vec,
        "edge_idx": edge_idx,
        "scaled_force": force,
    }


# speedup vs baseline: 2.3547x; 2.3547x over previous
import functools

import jax
import jax.numpy as jnp
from jax import lax
from jax.experimental import pallas as pl
from jax.experimental.pallas import tpu as pltpu

_LO = 256
_E_TILE = 4096
_N_PAR = 2


def _round_up(x, m):
    return ((x + m - 1) // m) * m


def _scatter_kernel(idx_ref, ev_ref, out_ref, *, hi_radix):
    ei = pl.program_id(1)

    @pl.when(ei == 0)
    def _init():
        out_ref[...] = jnp.zeros_like(out_ref)

    e_tile = ev_ref.shape[1]

    ev = ev_ref[...]
    d = ev + jnp.cos(ev)
    d16 = d.astype(jnp.bfloat16)

    hi_iota = lax.broadcasted_iota(jnp.int32, (hi_radix, e_tile), 0)
    lo_iota = lax.broadcasted_iota(jnp.int32, (_LO, e_tile), 0)

    accs = []
    for s in (0, 1):
        nid = idx_ref[s:s + 1, :]
        h = nid >> 8
        l = nid & (_LO - 1)
        hmask = (hi_iota == h).astype(jnp.bfloat16)
        lmask = (lo_iota == l).astype(jnp.bfloat16)
        dd = d16 if s == 0 else -d16
        p = jnp.concatenate(
            [hmask * dd[r:r + 1, :] for r in range(3)], axis=0)
        mm = lax.dot_general(
            p, lmask, (((1,), (1,)), ((), ())),
            preferred_element_type=jnp.float32)
        accs.append(mm)
    out_ref[...] += accs[0] + accs[1]


def kernel(node_feature, node_feature_ghost, edge_vec, edge_idx):
    tot_num = node_feature.shape[0] + node_feature_ghost.shape[0]
    num_edges = edge_vec.shape[0]

    n_pad = _round_up(max(tot_num, 1), _LO)
    hi_radix = n_pad // _LO

    e_tile = min(_E_TILE, _round_up(max(num_edges, 1), 256))
    e_pad = _round_up(max(num_edges, 1), _N_PAR * e_tile)
    e_blocks = e_pad // e_tile
    epc = e_blocks // _N_PAR

    if e_pad == num_edges:
        idx = edge_idx.astype(jnp.int32)
    else:
        idx = jnp.full((2, e_pad), -1, jnp.int32)
        idx = idx.at[:, :num_edges].set(edge_idx.astype(jnp.int32))

    if e_pad == num_edges:
        ev_t = edge_vec.T.astype(jnp.float32)
    else:
        ev_t = jnp.zeros((3, e_pad), jnp.float32)
        ev_t = ev_t.at[:, :num_edges].set(edge_vec.T.astype(jnp.float32))

    out = pl.pallas_call(
        functools.partial(_scatter_kernel, hi_radix=hi_radix),
        out_shape=jax.ShapeDtypeStruct((_N_PAR * 3 * hi_radix, _LO), jnp.float32),
        grid=(_N_PAR, epc),
        in_specs=[
            pl.BlockSpec((2, e_tile), lambda p, ei: (0, p * epc + ei)),
            pl.BlockSpec((3, e_tile), lambda p, ei: (0, p * epc + ei)),
        ],
        out_specs=pl.BlockSpec((3 * hi_radix, _LO), lambda p, ei: (p, 0)),
        compiler_params=pltpu.CompilerParams(
            dimension_semantics=("parallel", "arbitrary"),
            vmem_limit_bytes=64 * 1024 * 1024,
        ),
        cost_estimate=pl.CostEstimate(
            flops=2 * 2 * 2 * 3 * hi_radix * _LO * e_pad,
            transcendentals=3 * e_pad,
            bytes_accessed=(idx.size + ev_t.size) * 4 + 2 * 3 * n_pad * 4,
        ),
    )(idx, ev_t)

    force_t = out.reshape(_N_PAR, 3, hi_radix, _LO).sum(axis=0)
    force = force_t.transpose(1, 2, 0).reshape(n_pad, 3)[:tot_num]

    return {
        "node_feature": node_feature,
        "node_feature_ghost": node_feature_ghost,
        "edge_vec": edge_vec,
        "edge_idx": edge_idx,
        "scaled_force": force,
    }


# speedup vs baseline: 2.5159x; 1.0685x over previous
import functools

import jax
import jax.numpy as jnp
from jax import lax
from jax.experimental import pallas as pl
from jax.experimental.pallas import tpu as pltpu

_LO = 256
_E_TILE = 8192
_N_PAR = 2


def _round_up(x, m):
    return ((x + m - 1) // m) * m


def _scatter_kernel(idx_ref, ev_ref, out_ref, *, hi_radix):
    ei = pl.program_id(1)

    @pl.when(ei == 0)
    def _init():
        out_ref[...] = jnp.zeros_like(out_ref)

    e_tile = ev_ref.shape[1]

    ev = ev_ref[...]
    d = ev + jnp.cos(ev)
    d16 = d.astype(jnp.bfloat16)

    hi_iota = lax.broadcasted_iota(jnp.int32, (hi_radix, e_tile), 0)
    lo_iota = lax.broadcasted_iota(jnp.int32, (_LO, e_tile), 0)

    accs = []
    for s in (0, 1):
        nid = idx_ref[s:s + 1, :]
        h = nid >> 8
        l = nid & (_LO - 1)
        hmask = (hi_iota == h).astype(jnp.bfloat16)
        lmask = (lo_iota == l).astype(jnp.bfloat16)
        dd = d16 if s == 0 else -d16
        p = jnp.concatenate(
            [hmask * dd[r:r + 1, :] for r in range(3)], axis=0)
        mm = lax.dot_general(
            p, lmask, (((1,), (1,)), ((), ())),
            preferred_element_type=jnp.float32)
        accs.append(mm)
    out_ref[...] += accs[0] + accs[1]


def kernel(node_feature, node_feature_ghost, edge_vec, edge_idx):
    tot_num = node_feature.shape[0] + node_feature_ghost.shape[0]
    num_edges = edge_vec.shape[0]

    n_pad = _round_up(max(tot_num, 1), _LO)
    hi_radix = n_pad // _LO

    e_tile = min(_E_TILE, _round_up(max(num_edges, 1), 256))
    e_pad = _round_up(max(num_edges, 1), _N_PAR * e_tile)
    e_blocks = e_pad // e_tile
    epc = e_blocks // _N_PAR

    if e_pad == num_edges:
        idx = edge_idx.astype(jnp.int32)
    else:
        idx = jnp.full((2, e_pad), -1, jnp.int32)
        idx = idx.at[:, :num_edges].set(edge_idx.astype(jnp.int32))

    if e_pad == num_edges:
        ev_t = edge_vec.T.astype(jnp.float32)
    else:
        ev_t = jnp.zeros((3, e_pad), jnp.float32)
        ev_t = ev_t.at[:, :num_edges].set(edge_vec.T.astype(jnp.float32))

    out = pl.pallas_call(
        functools.partial(_scatter_kernel, hi_radix=hi_radix),
        out_shape=jax.ShapeDtypeStruct((_N_PAR * 3 * hi_radix, _LO), jnp.float32),
        grid=(_N_PAR, epc),
        in_specs=[
            pl.BlockSpec((2, e_tile), lambda p, ei: (0, p * epc + ei)),
            pl.BlockSpec((3, e_tile), lambda p, ei: (0, p * epc + ei)),
        ],
        out_specs=pl.BlockSpec((3 * hi_radix, _LO), lambda p, ei: (p, 0)),
        compiler_params=pltpu.CompilerParams(
            dimension_semantics=("parallel", "arbitrary"),
            vmem_limit_bytes=64 * 1024 * 1024,
        ),
        cost_estimate=pl.CostEstimate(
            flops=2 * 2 * 2 * 3 * hi_radix * _LO * e_pad,
            transcendentals=3 * e_pad,
            bytes_accessed=(idx.size + ev_t.size) * 4 + 2 * 3 * n_pad * 4,
        ),
    )(idx, ev_t)

    force_t = out.reshape(_N_PAR, 3, hi_radix, _LO).sum(axis=0)
    force = force_t.transpose(1, 2, 0).reshape(n_pad, 3)[:tot_num]

    return {
        "node_feature": node_feature,
        "node_feature_ghost": node_feature_ghost,
        "edge_vec": edge_vec,
        "edge_idx": edge_idx,
        "scaled_force": force,
    }


# speedup vs baseline: 2.6037x; 1.0349x over previous
import functools

import jax
import jax.numpy as jnp
from jax import lax
from jax.experimental import pallas as pl
from jax.experimental.pallas import tpu as pltpu

_LO = 256
_E_TILE = 16384
_N_PAR = 2


def _round_up(x, m):
    return ((x + m - 1) // m) * m


def _scatter_kernel(idx_ref, ev_ref, out_ref, *, hi_radix):
    ei = pl.program_id(1)

    @pl.when(ei == 0)
    def _init():
        out_ref[...] = jnp.zeros_like(out_ref)

    e_tile = ev_ref.shape[1]

    ev = ev_ref[...]
    d = ev + jnp.cos(ev)
    d16 = d.astype(jnp.bfloat16)

    hi_iota = lax.broadcasted_iota(jnp.int32, (hi_radix, e_tile), 0)
    lo_iota = lax.broadcasted_iota(jnp.int32, (_LO, e_tile), 0)

    accs = []
    for s in (0, 1):
        nid = idx_ref[s:s + 1, :]
        h = nid >> 8
        l = nid & (_LO - 1)
        hmask = (hi_iota == h).astype(jnp.bfloat16)
        lmask = (lo_iota == l).astype(jnp.bfloat16)
        dd = d16 if s == 0 else -d16
        p = jnp.concatenate(
            [hmask * dd[r:r + 1, :] for r in range(3)], axis=0)
        mm = lax.dot_general(
            p, lmask, (((1,), (1,)), ((), ())),
            preferred_element_type=jnp.float32)
        accs.append(mm)
    out_ref[...] += accs[0] + accs[1]


def kernel(node_feature, node_feature_ghost, edge_vec, edge_idx):
    tot_num = node_feature.shape[0] + node_feature_ghost.shape[0]
    num_edges = edge_vec.shape[0]

    n_pad = _round_up(max(tot_num, 1), _LO)
    hi_radix = n_pad // _LO

    e_tile = min(_E_TILE, _round_up(max(num_edges, 1), 256))
    e_pad = _round_up(max(num_edges, 1), _N_PAR * e_tile)
    e_blocks = e_pad // e_tile
    epc = e_blocks // _N_PAR

    if e_pad == num_edges:
        idx = edge_idx.astype(jnp.int32)
    else:
        idx = jnp.full((2, e_pad), -1, jnp.int32)
        idx = idx.at[:, :num_edges].set(edge_idx.astype(jnp.int32))

    if e_pad == num_edges:
        ev_t = edge_vec.T.astype(jnp.float32)
    else:
        ev_t = jnp.zeros((3, e_pad), jnp.float32)
        ev_t = ev_t.at[:, :num_edges].set(edge_vec.T.astype(jnp.float32))

    out = pl.pallas_call(
        functools.partial(_scatter_kernel, hi_radix=hi_radix),
        out_shape=jax.ShapeDtypeStruct((_N_PAR * 3 * hi_radix, _LO), jnp.float32),
        grid=(_N_PAR, epc),
        in_specs=[
            pl.BlockSpec((2, e_tile), lambda p, ei: (0, p * epc + ei)),
            pl.BlockSpec((3, e_tile), lambda p, ei: (0, p * epc + ei)),
        ],
        out_specs=pl.BlockSpec((3 * hi_radix, _LO), lambda p, ei: (p, 0)),
        compiler_params=pltpu.CompilerParams(
            dimension_semantics=("parallel", "arbitrary"),
            vmem_limit_bytes=64 * 1024 * 1024,
        ),
        cost_estimate=pl.CostEstimate(
            flops=2 * 2 * 2 * 3 * hi_radix * _LO * e_pad,
            transcendentals=3 * e_pad,
            bytes_accessed=(idx.size + ev_t.size) * 4 + 2 * 3 * n_pad * 4,
        ),
    )(idx, ev_t)

    force_t = out.reshape(_N_PAR, 3, hi_radix, _LO).sum(axis=0)
    force = force_t.transpose(1, 2, 0).reshape(n_pad, 3)[:tot_num]

    return {
        "node_feature": node_feature,
        "node_feature_ghost": node_feature_ghost,
        "edge_vec": edge_vec,
        "edge_idx": edge_idx,
        "scaled_force": force,
    }


# speedup vs baseline: 2.6464x; 1.0164x over previous
import functools

import jax
import jax.numpy as jnp
from jax import lax
from jax.experimental import pallas as pl
from jax.experimental.pallas import tpu as pltpu

_LO = 256
_E_TILE = 32768
_N_PAR = 2


def _round_up(x, m):
    return ((x + m - 1) // m) * m


def _scatter_kernel(idx_ref, ev_ref, out_ref, *, hi_radix):
    ei = pl.program_id(1)

    @pl.when(ei == 0)
    def _init():
        out_ref[...] = jnp.zeros_like(out_ref)

    e_tile = ev_ref.shape[1]

    ev = ev_ref[...]
    d = ev + jnp.cos(ev)
    d16 = d.astype(jnp.bfloat16)

    hi_iota = lax.broadcasted_iota(jnp.int32, (hi_radix, e_tile), 0)
    lo_iota = lax.broadcasted_iota(jnp.int32, (_LO, e_tile), 0)

    accs = []
    for s in (0, 1):
        nid = idx_ref[s:s + 1, :]
        h = nid >> 8
        l = nid & (_LO - 1)
        hmask = (hi_iota == h).astype(jnp.bfloat16)
        lmask = (lo_iota == l).astype(jnp.bfloat16)
        dd = d16 if s == 0 else -d16
        p = jnp.concatenate(
            [hmask * dd[r:r + 1, :] for r in range(3)], axis=0)
        mm = lax.dot_general(
            p, lmask, (((1,), (1,)), ((), ())),
            preferred_element_type=jnp.float32)
        accs.append(mm)
    out_ref[...] += accs[0] + accs[1]


def kernel(node_feature, node_feature_ghost, edge_vec, edge_idx):
    tot_num = node_feature.shape[0] + node_feature_ghost.shape[0]
    num_edges = edge_vec.shape[0]

    n_pad = _round_up(max(tot_num, 1), _LO)
    hi_radix = n_pad // _LO

    e_tile = min(_E_TILE, _round_up(max(num_edges, 1), 256))
    e_pad = _round_up(max(num_edges, 1), _N_PAR * e_tile)
    e_blocks = e_pad // e_tile
    epc = e_blocks // _N_PAR

    if e_pad == num_edges:
        idx = edge_idx.astype(jnp.int32)
    else:
        idx = jnp.full((2, e_pad), -1, jnp.int32)
        idx = idx.at[:, :num_edges].set(edge_idx.astype(jnp.int32))

    if e_pad == num_edges:
        ev_t = edge_vec.T.astype(jnp.float32)
    else:
        ev_t = jnp.zeros((3, e_pad), jnp.float32)
        ev_t = ev_t.at[:, :num_edges].set(edge_vec.T.astype(jnp.float32))

    out = pl.pallas_call(
        functools.partial(_scatter_kernel, hi_radix=hi_radix),
        out_shape=jax.ShapeDtypeStruct((_N_PAR * 3 * hi_radix, _LO), jnp.float32),
        grid=(_N_PAR, epc),
        in_specs=[
            pl.BlockSpec((2, e_tile), lambda p, ei: (0, p * epc + ei)),
            pl.BlockSpec((3, e_tile), lambda p, ei: (0, p * epc + ei)),
        ],
        out_specs=pl.BlockSpec((3 * hi_radix, _LO), lambda p, ei: (p, 0)),
        compiler_params=pltpu.CompilerParams(
            dimension_semantics=("parallel", "arbitrary"),
            vmem_limit_bytes=64 * 1024 * 1024,
        ),
        cost_estimate=pl.CostEstimate(
            flops=2 * 2 * 2 * 3 * hi_radix * _LO * e_pad,
            transcendentals=3 * e_pad,
            bytes_accessed=(idx.size + ev_t.size) * 4 + 2 * 3 * n_pad * 4,
        ),
    )(idx, ev_t)

    force_t = out.reshape(_N_PAR, 3, hi_radix, _LO).sum(axis=0)
    force = force_t.transpose(1, 2, 0).reshape(n_pad, 3)[:tot_num]

    return {
        "node_feature": node_feature,
        "node_feature_ghost": node_feature_ghost,
        "edge_vec": edge_vec,
        "edge_idx": edge_idx,
        "scaled_force": force,
    }
